# baseline (speedup 1.0000x reference)
import functools

import jax
import jax.numpy as jnp
from jax import lax
from jax.experimental import pallas as pl
from jax.experimental.pallas import tpu as pltpu

ACT = 1024
DICT = 16384
TOKENS = 4096
TOPK = 64
L1_COEFF = 0.0008

TB = 512
DC = 512
ND = DICT // DC
TBLKS = TOKENS // TB


def _norm_stats(xb):
    mean = jnp.mean(xb, axis=1, keepdims=True)
    xc = xb - mean
    var = jnp.sum(xc * xc, axis=1, keepdims=True) * (1.0 / (ACT - 1))
    std = jnp.sqrt(var)
    xn = xc / (std + 1e-5)
    return mean, std, xn


def _sae_kernel(x_ref, weh_ref, wel_ref, wdh_ref, wdl_ref, b_dec_ref,
                sae_out_ref, acts_topk_ref, sae_err_ref, part_ref,
                acts_s, xeh_s, xel_s, misc_s):
    s = pl.program_id(1)

    @pl.when(s == 0)
    def _normalize():
        _, _, xn = _norm_stats(x_ref[...])
        xe = xn - b_dec_ref[...]
        xeh = xe.astype(jnp.bfloat16)
        xeh_s[...] = xeh
        xel_s[...] = (xe - xeh.astype(jnp.float32)).astype(jnp.bfloat16)

    @pl.when(s < ND)
    def _encode():
        xeh = xeh_s[...]
        xel = xel_s[...]
        weh = weh_ref[...]
        z = (jnp.dot(xeh, weh, preferred_element_type=jnp.float32)
             + jnp.dot(xeh, wel_ref[...], preferred_element_type=jnp.float32)
             + jnp.dot(xel, weh, preferred_element_type=jnp.float32))
        acts_s[:, pl.ds(s * DC, DC)] = jnp.maximum(z, 0.0)

    @pl.when(s == ND)
    def _threshold():
        def body(i, lo):
            bitpos = 30 - i
            t = lo | jnp.left_shift(jnp.int32(1), bitpos)
            bits = lax.bitcast_convert_type(acts_s[...], jnp.int32)
            cnt = jnp.sum((bits >= t).astype(jnp.int32), axis=1, keepdims=True)
            return jnp.where(cnt >= TOPK, t, lo)

        lo = jnp.zeros((TB, 1), jnp.int32)
        lo = lax.fori_loop(0, 31, body, lo)
        misc_s[:, 0:1] = lax.bitcast_convert_type(lo, jnp.float32)

    @pl.when(s >= ND)
    def _mask_decode():
        c = s - ND
        acts = acts_s[:, pl.ds(c * DC, DC)]
        atk = jnp.where(acts >= misc_s[:, 0:1], acts, 0.0)
        acts_topk_ref[...] = atk
        ah = atk.astype(jnp.bfloat16)
        al = (atk - ah.astype(jnp.float32)).astype(jnp.bfloat16)
        wdh = wdh_ref[...]
        part = (jnp.dot(ah, wdh, preferred_element_type=jnp.float32)
                + jnp.dot(ah, wdl_ref[...], preferred_element_type=jnp.float32)
                + jnp.dot(al, wdh, preferred_element_type=jnp.float32))
        l1c = jnp.sum(atk, axis=1, keepdims=True)
        l0c = jnp.sum((atk > 0).astype(jnp.float32), axis=1, keepdims=True)

        @pl.when(s == ND)
        def _init():
            sae_out_ref[...] = part
            misc_s[:, 1:2] = l1c
            misc_s[:, 2:3] = l0c

        @pl.when(s > ND)
        def _acc():
            sae_out_ref[...] = sae_out_ref[...] + part
            misc_s[:, 1:2] = misc_s[:, 1:2] + l1c
            misc_s[:, 2:3] = misc_s[:, 2:3] + l0c

    @pl.when(s == 2 * ND - 1)
    def _finalize():
        mean, std, xn = _norm_stats(x_ref[...])
        xrec = sae_out_ref[...] + b_dec_ref[...]
        sae_out = xrec * std + mean
        sae_out_ref[...] = sae_out
        sae_err_ref[...] = (xn * std + mean) - sae_out
        diff = xrec - xn
        l2p = jnp.sum(diff * diff)
        l1p = jnp.sum(misc_s[:, 1:2])
        l0p = jnp.sum(misc_s[:, 2:3])
        lane = lax.broadcasted_iota(jnp.int32, (1, 1, 128), 2)
        part_ref[...] = jnp.where(
            lane == 0, l2p, jnp.where(lane == 1, l1p,
                                      jnp.where(lane == 2, l0p, 0.0)))


@functools.partial(jax.jit)
def _run(xs, weh, wel, wdh, wdl, b_dec2):
    grid = (TBLKS, 2 * ND)
    out_shapes = (
        jax.ShapeDtypeStruct((TOKENS, ACT), jnp.float32),
        jax.ShapeDtypeStruct((TOKENS, DICT), jnp.float32),
        jax.ShapeDtypeStruct((TOKENS, ACT), jnp.float32),
        jax.ShapeDtypeStruct((TBLKS, 1, 128), jnp.float32),
    )
    enc_ix = lambda t, s: (0, jnp.minimum(s, ND - 1))
    dec_ix = lambda t, s: (jnp.maximum(s - ND, 0), 0)
    in_specs = [
        pl.BlockSpec((TB, ACT), lambda t, s: (t, 0)),
        pl.BlockSpec((ACT, DC), enc_ix),
        pl.BlockSpec((ACT, DC), enc_ix),
        pl.BlockSpec((DC, ACT), dec_ix),
        pl.BlockSpec((DC, ACT), dec_ix),
        pl.BlockSpec((1, ACT), lambda t, s: (0, 0)),
    ]
    out_specs = (
        pl.BlockSpec((TB, ACT), lambda t, s: (t, 0)),
        pl.BlockSpec((TB, DC), lambda t, s: (t, jnp.maximum(s - ND, 0))),
        pl.BlockSpec((TB, ACT), lambda t, s: (t, 0)),
        pl.BlockSpec((1, 1, 128), lambda t, s: (t, 0, 0)),
    )
    scratch = [
        pltpu.VMEM((TB, DICT), jnp.float32),
        pltpu.VMEM((TB, ACT), jnp.bfloat16),
        pltpu.VMEM((TB, ACT), jnp.bfloat16),
        pltpu.VMEM((TB, 128), jnp.float32),
    ]
    return pl.pallas_call(
        _sae_kernel,
        grid=grid,
        in_specs=in_specs,
        out_specs=out_specs,
        out_shape=out_shapes,
        scratch_shapes=scratch,
        compiler_params=pltpu.CompilerParams(
            dimension_semantics=("arbitrary", "arbitrary"),
        ),
    )(xs, weh, wel, wdh, wdl, b_dec2)


def kernel(x, W_enc, W_dec, b_dec):
    xs = x[0]
    b_dec2 = b_dec.reshape(1, ACT)
    weh = W_enc.astype(jnp.bfloat16)
    wel = (W_enc - weh.astype(jnp.float32)).astype(jnp.bfloat16)
    wdh = W_dec.astype(jnp.bfloat16)
    wdl = (W_dec - wdh.astype(jnp.float32)).astype(jnp.bfloat16)
    sae_out, acts_topk, sae_error, parts = _run(xs, weh, wel, wdh, wdl, b_dec2)
    l2_sum = jnp.sum(parts[:, 0, 0])
    l1_sum = jnp.sum(parts[:, 0, 1])
    l0_sum = jnp.sum(parts[:, 0, 2])
    l2_loss = l2_sum / (TOKENS * ACT)
    l1_norm = l1_sum / TOKENS
    l0_norm = l0_sum / TOKENS
    l1_loss = L1_COEFF * l1_norm
    loss = l2_loss
    return sae_out, acts_topk, loss, l1_loss, l2_loss, l0_norm, l1_norm, sae_error


# speedup vs baseline: 1.3290x; 1.3290x over previous
import functools

import jax
import jax.numpy as jnp
from jax import lax
from jax.experimental import pallas as pl
from jax.experimental.pallas import tpu as pltpu

ACT = 1024
DICT = 16384
TOKENS = 4096
TOPK = 64
L1_COEFF = 0.0008

TB = 512
DC = 512
ND = DICT // DC
TBLKS = TOKENS // TB


def _norm_stats(xb):
    mean = jnp.mean(xb, axis=1, keepdims=True)
    xc = xb - mean
    var = jnp.sum(xc * xc, axis=1, keepdims=True) * (1.0 / (ACT - 1))
    std = jnp.sqrt(var)
    xn = xc / (std + 1e-5)
    return mean, std, xn


def _sae_kernel(x_ref, w_enc_ref, w_dec_ref, b_dec_ref,
                sae_out_ref, acts_topk_ref, sae_err_ref, part_ref,
                acts_s, misc_s, wd_buf, wd_sem):
    s = pl.program_id(1)

    @pl.when(s == 0)
    def _normalize():
        _, _, xn = _norm_stats(x_ref[...])
        sae_err_ref[...] = xn - b_dec_ref[...]

    @pl.when(s < ND)
    def _encode():
        z = jnp.dot(sae_err_ref[...], w_enc_ref[...],
                    preferred_element_type=jnp.float32)
        acts_s[:, pl.ds(s * DC, DC)] = jnp.maximum(z, 0.0)

    @pl.when(s == ND)
    def _threshold():
        def body(i, lo):
            bitpos = 30 - i
            t = lo | jnp.left_shift(jnp.int32(1), bitpos)
            bits = lax.bitcast_convert_type(acts_s[...], jnp.int32)
            cnt = jnp.sum((bits >= t).astype(jnp.int32), axis=1, keepdims=True)
            return jnp.where(cnt >= TOPK, t, lo)

        lo = jnp.zeros((TB, 1), jnp.int32)
        lo = lax.fori_loop(0, 31, body, lo)
        misc_s[:, 0:1] = lax.bitcast_convert_type(lo, jnp.float32)
        for c in range(2):
            pltpu.make_async_copy(
                w_dec_ref.at[pl.ds(c * DC, DC), :], wd_buf.at[c],
                wd_sem.at[c]).start()

    @pl.when(s >= ND)
    def _mask_decode():
        c = s - ND
        acts = acts_s[:, pl.ds(c * DC, DC)]
        atk = jnp.where(acts >= misc_s[:, 0:1], acts, 0.0)
        acts_topk_ref[...] = atk
        slot = lax.rem(c, 2)
        pltpu.make_async_copy(
            w_dec_ref.at[pl.ds(c * DC, DC), :], wd_buf.at[slot],
            wd_sem.at[slot]).wait()
        part = jnp.dot(atk, wd_buf[slot], preferred_element_type=jnp.float32)

        @pl.when(c + 2 < ND)
        def _prefetch():
            nslot = lax.rem(c + 2, 2)
            pltpu.make_async_copy(
                w_dec_ref.at[pl.ds((c + 2) * DC, DC), :], wd_buf.at[nslot],
                wd_sem.at[nslot]).start()
        l1c = jnp.sum(atk, axis=1, keepdims=True)
        l0c = jnp.sum((atk > 0).astype(jnp.float32), axis=1, keepdims=True)

        @pl.when(s == ND)
        def _init():
            sae_out_ref[...] = part
            misc_s[:, 1:2] = l1c
            misc_s[:, 2:3] = l0c

        @pl.when(s > ND)
        def _acc():
            sae_out_ref[...] = sae_out_ref[...] + part
            misc_s[:, 1:2] = misc_s[:, 1:2] + l1c
            misc_s[:, 2:3] = misc_s[:, 2:3] + l0c

    @pl.when(s == 2 * ND - 1)
    def _finalize():
        mean, std, xn = _norm_stats(x_ref[...])
        xrec = sae_out_ref[...] + b_dec_ref[...]
        sae_out = xrec * std + mean
        sae_out_ref[...] = sae_out
        sae_err_ref[...] = (xn * std + mean) - sae_out
        diff = xrec - xn
        l2p = jnp.sum(diff * diff)
        l1p = jnp.sum(misc_s[:, 1:2])
        l0p = jnp.sum(misc_s[:, 2:3])
        lane = lax.broadcasted_iota(jnp.int32, (1, 1, 128), 2)
        part_ref[...] = jnp.where(
            lane == 0, l2p, jnp.where(lane == 1, l1p,
                                      jnp.where(lane == 2, l0p, 0.0)))


@functools.partial(jax.jit)
def _run(xs, W_enc, W_dec, b_dec2):
    grid = (TBLKS, 2 * ND)
    out_shapes = (
        jax.ShapeDtypeStruct((TOKENS, ACT), jnp.float32),
        jax.ShapeDtypeStruct((TOKENS, DICT), jnp.float32),
        jax.ShapeDtypeStruct((TOKENS, ACT), jnp.float32),
        jax.ShapeDtypeStruct((TBLKS, 1, 128), jnp.float32),
    )
    in_specs = [
        pl.BlockSpec((TB, ACT), lambda t, s: (t, 0)),
        pl.BlockSpec((ACT, DC), lambda t, s: (0, jnp.minimum(s, ND - 1))),
        pl.BlockSpec(memory_space=pl.ANY),
        pl.BlockSpec((1, ACT), lambda t, s: (0, 0)),
    ]
    out_specs = (
        pl.BlockSpec((TB, ACT), lambda t, s: (t, 0)),
        pl.BlockSpec((TB, DC), lambda t, s: (t, jnp.maximum(s - ND, 0))),
        pl.BlockSpec((TB, ACT), lambda t, s: (t, 0)),
        pl.BlockSpec((1, 1, 128), lambda t, s: (t, 0, 0)),
    )
    scratch = [
        pltpu.VMEM((TB, DICT), jnp.float32),
        pltpu.VMEM((TB, 128), jnp.float32),
        pltpu.VMEM((2, DC, ACT), jnp.float32),
        pltpu.SemaphoreType.DMA((2,)),
    ]
    return pl.pallas_call(
        _sae_kernel,
        grid=grid,
        in_specs=in_specs,
        out_specs=out_specs,
        out_shape=out_shapes,
        scratch_shapes=scratch,
        compiler_params=pltpu.CompilerParams(
            dimension_semantics=("arbitrary", "arbitrary"),
        ),
    )(xs, W_enc, W_dec, b_dec2)


def kernel(x, W_enc, W_dec, b_dec):
    xs = x[0]
    b_dec2 = b_dec.reshape(1, ACT)
    sae_out, acts_topk, sae_error, parts = _run(xs, W_enc, W_dec, b_dec2)
    l2_sum = jnp.sum(parts[:, 0, 0])
    l1_sum = jnp.sum(parts[:, 0, 1])
    l0_sum = jnp.sum(parts[:, 0, 2])
    l2_loss = l2_sum / (TOKENS * ACT)
    l1_norm = l1_sum / TOKENS
    l0_norm = l0_sum / TOKENS
    l1_loss = L1_COEFF * l1_norm
    loss = l2_loss
    return sae_out, acts_topk, loss, l1_loss, l2_loss, l0_norm, l1_norm, sae_error
